# baseline (speedup 1.0000x reference)
import functools

import jax
import jax.numpy as jnp
from jax import lax
from jax.experimental import pallas as pl
from jax.experimental.pallas import tpu as pltpu
from jax.experimental.pallas import tpu_sc as plsc

VOCAB = 100000
NGRAM = 100000
HID = 64
B = 4096

NC, NS = 2, 16
NW = NC * NS
EPW = B // NW
G = 8
NGROUP = EPW // G
NP, NN = 5, 20
NL = NP + NN
LANES = 16


def _log_sigmoid(x):
    u = jnp.exp(-jnp.abs(x))
    s = u / (u + 2.0)
    s2 = s * s
    p = jnp.float32(1.0 / 11.0)
    for c in (1.0 / 9.0, 1.0 / 7.0, 1.0 / 5.0, 1.0 / 3.0, 1.0):
        p = p * s2 + jnp.float32(c)
    return jnp.minimum(x, 0.0) - 2.0 * s * p


NBLK = (VOCAB // 2) // HID
LINES_PAD = VOCAB // 2 + 48


def _transpose_body(tt_in, tt_ng, tl_in, tl_ng,
                    o_in, o_ng,
                    vbuf, obuf, sin, sout):
    wid = lax.axis_index("s") * NC + lax.axis_index("c")
    iota = lax.iota(jnp.int32, LANES)
    nblk = (NBLK - wid + NW - 1) // NW

    def mo(x, m=8):
        return pl.multiple_of(x, m)

    def transpose_block(buf, ncs):
        rows_h = [iota + hs * 16 for hs in range(HID // 16)]
        src = vbuf.at[buf]
        dst = obuf.at[buf]

        def dloop(d4, carry):
            for du in range(4):
                d = d4 * 4 + du
                perm = (iota + d) & 15
                ph = perm >> 1
                pp = (perm & 1) << 6
                for hp in range(HID // 32):
                    batch = []
                    for hs in (2 * hp, 2 * hp + 1):
                        col_s = pp + rows_h[hs]
                        for cs in range(ncs):
                            cols = perm + (cs * 16)
                            q = ph + (cs * 8)
                            batch.append(
                                (plsc.load_gather(src, [rows_h[hs], cols]),
                                 q, col_s))
                    for v, q, col_s in batch:
                        plsc.store_scatter(dst, [q, col_s], v)
            return carry

        lax.fori_loop(0, LANES // 4, dloop, 0)

    for tt, tl, out in ((tt_in, tl_in, o_in), (tt_ng, tl_ng, o_ng)):
        @pl.when(wid == 13)
        def _():
            pltpu.sync_copy(tl, vbuf.at[0, pl.ds(0, 16)])
            pltpu.sync_copy(vbuf.at[0, pl.ds(0, 16)],
                            out.at[pl.ds(NBLK * 64, 16)])

        def stage(i):
            b = wid + NW * i
            return pltpu.make_async_copy(
                tt.at[:, pl.ds(mo(b * 128, 128), 128)],
                vbuf.at[i & 3], sin)

        def ocopy(i):
            b = wid + NW * i
            return pltpu.make_async_copy(
                obuf.at[i & 3], out.at[pl.ds(mo(b * 64), 64)], sout)

        def bloop(i, carry):
            @pl.when(wid + NW * i < NBLK)
            def _():
                @pl.when(i == 0)
                def _():
                    for k in (0, 1, 2):
                        @pl.when(wid + NW * k < NBLK)
                        def _():
                            stage(k).start()
                stage(i).wait()

                @pl.when(wid + NW * (i + 3) < NBLK)
                def _():
                    stage(i + 3).start()

                @pl.when(i >= 4)
                def _():
                    ocopy(i - 4).wait()
                transpose_block(i & 3, 2 * HID // 16)
                ocopy(i).start()
            return carry

        niter = (NBLK + NW - 1) // NW
        lax.fori_loop(0, niter, bloop, 0)
        for j in (4, 3, 2, 1):
            @pl.when(nblk >= j)
            def _():
                ocopy(nblk - j).wait()


def _sc_body(in_idx_hbm, ng_idx_hbm, pn_idx_hbm, in_emb, ngram_emb, out_emb,
             out_hbm,
             in_idx_v, ng_idx_v, pn_idx_v, in_ln, ng_ln, pn_ln,
             in_rows, ng_rows, out_rows, loss_v, sem):
    wid = lax.axis_index("s") * NC + lax.axis_index("c")
    base = wid * EPW

    pltpu.sync_copy(in_idx_hbm.at[pl.ds(base, EPW)],
                    in_idx_v.at[pl.ds(0, EPW)])
    pltpu.sync_copy(ng_idx_hbm.at[pl.ds(base * 6, EPW * 6)],
                    ng_idx_v.at[pl.ds(0, EPW * 6)])
    pltpu.sync_copy(pn_idx_hbm.at[pl.ds(base * NL, EPW * NL)],
                    pn_idx_v.at[pl.ds(0, EPW * NL)])

    def shift_into(src, dst, nvec):
        def body(i, carry):
            dst[pl.ds(i * LANES, LANES)] = src[pl.ds(i * LANES, LANES)] >> 1
            return carry
        lax.fori_loop(0, nvec, body, 0)

    shift_into(in_idx_v, in_ln, EPW // LANES)
    shift_into(ng_idx_v, ng_ln, EPW * 6 // LANES)
    shift_into(pn_idx_v, pn_ln, EPW * NL // LANES)

    def mo(x):
        return pl.multiple_of(x, 8)

    def group_copies(g, buf):
        cps = [
            pltpu.make_async_copy(in_emb.at[in_ln.at[pl.ds(mo(g * G), G)]],
                                  in_rows.at[buf], sem),
            pltpu.make_async_copy(
                ngram_emb.at[ng_ln.at[pl.ds(mo(g * G * 6), G * 6)]],
                ng_rows.at[buf], sem),
        ]
        for c in range(5):
            cps.append(pltpu.make_async_copy(
                out_emb.at[pn_ln.at[pl.ds(mo(g * G * NL + c * 40), 40)]],
                out_rows.at[buf, pl.ds(c * 40, 40)], sem))
        return cps

    def start_group(g, buf):
        for c in group_copies(g, buf):
            c.start()

    def wait_group(g, buf):
        for c in group_copies(g, buf):
            c.wait()

    iota = lax.iota(jnp.int32, LANES)

    def compute_group(g, buf, lv):
        def ebody(e, lv):
            ge = g * G + e
            iv = in_idx_v[pl.ds(ge, 16)]
            nv = ng_idx_v[pl.ds(ge * 6, 16)]
            pv0 = pn_idx_v[pl.ds(ge * NL, 16)]
            pv1 = pn_idx_v[pl.ds(ge * NL + 9, 16)]
            off_in = (iv[0] & 1) << 6
            hs = []
            offs_ng = [(nv[q] & 1) << 6 for q in range(6)]
            for k in range(HID // 16):
                acc = ng_rows[buf, e * 6, pl.ds(offs_ng[0] + k * 16, 16)]
                for q in range(1, 6):
                    acc = acc + ng_rows[buf, e * 6 + q,
                                        pl.ds(offs_ng[q] + k * 16, 16)]
                hs.append((in_rows[buf, e, pl.ds(off_in + k * 16, 16)]
                           + acc * jnp.float32(1.0 / 6.0)) * jnp.float32(0.5))
            dv0 = jnp.full((LANES,), 30.0, jnp.float32)
            dv1 = dv0
            for j in range(NL):
                r = e * NL + j
                par = pv0[j] if j < 16 else pv1[j - 9]
                off = (par & 1) << 6
                t = out_rows[buf, r, pl.ds(off, 16)] * hs[0]
                for k in range(1, HID // 16):
                    t = t + out_rows[buf, r, pl.ds(off + k * 16, 16)] * hs[k]
                d = jnp.sum(t)
                db = jnp.full((LANES,), d if j < NP else -d)
                if j < LANES:
                    dv0 = jnp.where(iota == j, db, dv0)
                else:
                    dv1 = jnp.where(iota == (j - LANES), db, dv1)
            l = jnp.sum(_log_sigmoid(dv0)) + jnp.sum(_log_sigmoid(dv1))
            lane = (g % 2) * G + e
            return jnp.where(iota == lane, jnp.full((LANES,), -l), lv)

        return lax.fori_loop(0, G, ebody, lv)

    start_group(0, 0)

    def gloop(g, lv):
        buf = g & 1
        wait_group(g, buf)

        @pl.when(g < NGROUP - 1)
        def _():
            start_group(g + 1, 1 - buf)

        lv = compute_group(g, buf, lv)
        odd = (g & 1) == 1

        @pl.when(odd)
        def _():
            loss_v[pl.ds(mo((g - 1) * G), LANES)] = lv

        return jnp.where(jnp.full((LANES,), odd),
                         jnp.zeros((LANES,), jnp.float32), lv)

    lax.fori_loop(0, NGROUP, gloop, jnp.zeros((LANES,), jnp.float32))

    pltpu.sync_copy(loss_v, out_hbm.at[pl.ds(base, EPW)])


@jax.jit
def kernel(input_labels, pos_labels, neg_labels, ngram_labels,
           in_emb, ngram_emb, out_emb):
    in_idx = input_labels.astype(jnp.int32)
    ng_idx = ngram_labels.astype(jnp.int32).reshape(-1)
    pn_idx = jnp.concatenate(
        [pos_labels.astype(jnp.int32), neg_labels.astype(jnp.int32)],
        axis=1).reshape(-1)
    mesh = plsc.VectorSubcoreMesh(core_axis_name="c", subcore_axis_name="s")
    line_tab = jax.ShapeDtypeStruct((LINES_PAD, 2 * HID), jnp.float32)
    convert = functools.partial(
        pl.kernel,
        mesh=mesh,
        compiler_params=pltpu.CompilerParams(
            needs_layout_passes=False, use_tc_tiling_on_sc=True),
        out_type=(line_tab, line_tab),
        scratch_types=[
            pltpu.VMEM((4, HID, 2 * HID), jnp.float32),
            pltpu.VMEM((4, HID, 2 * HID), jnp.float32),
            pltpu.SemaphoreType.DMA,
            pltpu.SemaphoreType.DMA,
        ],
    )(_transpose_body)
    tails = [t[NBLK * 128:].reshape(16, 2 * HID)
             for t in (in_emb, ngram_emb)]
    in_t, ng_t = convert(in_emb.T, ngram_emb.T, *tails)
    out_t = out_emb.reshape(VOCAB // 2, 2 * HID)
    run = functools.partial(
        pl.kernel,
        mesh=mesh,
        compiler_params=pltpu.CompilerParams(
            needs_layout_passes=False, use_tc_tiling_on_sc=True),
        out_type=jax.ShapeDtypeStruct((B,), jnp.float32),
        scratch_types=[
            pltpu.VMEM((EPW + 16,), jnp.int32),
            pltpu.VMEM((EPW * 6 + 16,), jnp.int32),
            pltpu.VMEM((EPW * NL + 16,), jnp.int32),
            pltpu.VMEM((EPW,), jnp.int32),
            pltpu.VMEM((EPW * 6,), jnp.int32),
            pltpu.VMEM((EPW * NL,), jnp.int32),
            pltpu.VMEM((2, G, 2 * HID), jnp.float32),
            pltpu.VMEM((2, G * 6, 2 * HID), jnp.float32),
            pltpu.VMEM((2, G * NL, 2 * HID), jnp.float32),
            pltpu.VMEM((EPW,), jnp.float32),
            pltpu.SemaphoreType.DMA,
        ],
    )(_sc_body)
    return run(in_idx, ng_idx, pn_idx, in_t, ng_t, out_t)


# speedup vs baseline: 1.1910x; 1.1910x over previous
import functools

import jax
import jax.numpy as jnp
from jax import lax
from jax.experimental import pallas as pl
from jax.experimental.pallas import tpu as pltpu
from jax.experimental.pallas import tpu_sc as plsc

VOCAB = 100000
NGRAM = 100000
HID = 64
B = 4096

NC, NS = 2, 16
NW = NC * NS
EPW = B // NW
G = 8
NGROUP = EPW // G
NP, NN = 5, 20
NL = NP + NN
LANES = 16


def _log_sigmoid(x):
    u = jnp.exp(-jnp.abs(x))
    s = u / (u + 2.0)
    s2 = s * s
    p = jnp.float32(1.0 / 11.0)
    for c in (1.0 / 9.0, 1.0 / 7.0, 1.0 / 5.0, 1.0 / 3.0, 1.0):
        p = p * s2 + jnp.float32(c)
    return jnp.minimum(x, 0.0) - 2.0 * s * p


NBLK = (VOCAB // 2) // HID
LINES_PAD = VOCAB // 2 + 48


def _transpose_body(tt_in, tt_ng, tt_out, tl_in, tl_ng, tl_out,
                    o_in, o_ng, o_out,
                    vbuf, obuf, sin, sout):
    wid = lax.axis_index("s") * NC + lax.axis_index("c")
    iota = lax.iota(jnp.int32, LANES)
    nblk = (NBLK - wid + NW - 1) // NW

    def mo(x, m=8):
        return pl.multiple_of(x, m)

    def transpose_block(buf, ncs):
        rows_h = [iota + hs * 16 for hs in range(HID // 16)]
        src = vbuf.at[buf]
        dst = obuf.at[buf]

        def dloop(d4, carry):
            for du in range(4):
                d = d4 * 4 + du
                perm = (iota + d) & 15
                ph = perm >> 1
                pp = (perm & 1) << 6
                for hp in range(HID // 32):
                    batch = []
                    for hs in (2 * hp, 2 * hp + 1):
                        col_s = pp + rows_h[hs]
                        for cs in range(ncs):
                            cols = perm + (cs * 16)
                            q = ph + (cs * 8)
                            batch.append(
                                (plsc.load_gather(src, [rows_h[hs], cols]),
                                 q, col_s))
                    for v, q, col_s in batch:
                        plsc.store_scatter(dst, [q, col_s], v)
            return carry

        lax.fori_loop(0, LANES // 4, dloop, 0)

    for tt, tl, out in ((tt_in, tl_in, o_in), (tt_ng, tl_ng, o_ng),
                        (tt_out, tl_out, o_out)):
        @pl.when(wid == 13)
        def _():
            pltpu.sync_copy(tl, vbuf.at[0, pl.ds(0, 16)])
            pltpu.sync_copy(vbuf.at[0, pl.ds(0, 16)],
                            out.at[pl.ds(NBLK * 64, 16)])

        def stage(i):
            b = wid + NW * i
            return pltpu.make_async_copy(
                tt.at[:, pl.ds(mo(b * 128, 128), 128)],
                vbuf.at[i & 3], sin)

        def ocopy(i):
            b = wid + NW * i
            return pltpu.make_async_copy(
                obuf.at[i & 3], out.at[pl.ds(mo(b * 64), 64)], sout)

        def bloop(i, carry):
            @pl.when(wid + NW * i < NBLK)
            def _():
                @pl.when(i == 0)
                def _():
                    for k in (0, 1, 2):
                        @pl.when(wid + NW * k < NBLK)
                        def _():
                            stage(k).start()
                stage(i).wait()

                @pl.when(wid + NW * (i + 3) < NBLK)
                def _():
                    stage(i + 3).start()

                @pl.when(i >= 4)
                def _():
                    ocopy(i - 4).wait()
                transpose_block(i & 3, 2 * HID // 16)
                ocopy(i).start()
            return carry

        niter = (NBLK + NW - 1) // NW
        lax.fori_loop(0, niter, bloop, 0)
        for j in (4, 3, 2, 1):
            @pl.when(nblk >= j)
            def _():
                ocopy(nblk - j).wait()


def _sc_body(in_idx_hbm, ng_idx_hbm, pn_idx_hbm, in_emb, ngram_emb, out_emb,
             out_hbm,
             in_idx_v, ng_idx_v, pn_idx_v, in_ln, ng_ln, pn_ln,
             in_rows, ng_rows, out_rows, loss_v, sem):
    wid = lax.axis_index("s") * NC + lax.axis_index("c")
    base = wid * EPW

    pltpu.sync_copy(in_idx_hbm.at[pl.ds(base, EPW)],
                    in_idx_v.at[pl.ds(0, EPW)])
    pltpu.sync_copy(ng_idx_hbm.at[pl.ds(base * 6, EPW * 6)],
                    ng_idx_v.at[pl.ds(0, EPW * 6)])
    pltpu.sync_copy(pn_idx_hbm.at[pl.ds(base * NL, EPW * NL)],
                    pn_idx_v.at[pl.ds(0, EPW * NL)])

    def shift_into(src, dst, nvec):
        def body(i, carry):
            dst[pl.ds(i * LANES, LANES)] = src[pl.ds(i * LANES, LANES)] >> 1
            return carry
        lax.fori_loop(0, nvec, body, 0)

    shift_into(in_idx_v, in_ln, EPW // LANES)
    shift_into(ng_idx_v, ng_ln, EPW * 6 // LANES)
    shift_into(pn_idx_v, pn_ln, EPW * NL // LANES)

    def mo(x):
        return pl.multiple_of(x, 8)

    def group_copies(g, buf):
        cps = [
            pltpu.make_async_copy(in_emb.at[in_ln.at[pl.ds(mo(g * G), G)]],
                                  in_rows.at[buf], sem),
            pltpu.make_async_copy(
                ngram_emb.at[ng_ln.at[pl.ds(mo(g * G * 6), G * 6)]],
                ng_rows.at[buf], sem),
        ]
        for c in range(5):
            cps.append(pltpu.make_async_copy(
                out_emb.at[pn_ln.at[pl.ds(mo(g * G * NL + c * 40), 40)]],
                out_rows.at[buf, pl.ds(c * 40, 40)], sem))
        return cps

    def start_group(g, buf):
        for c in group_copies(g, buf):
            c.start()

    def wait_group(g, buf):
        for c in group_copies(g, buf):
            c.wait()

    iota = lax.iota(jnp.int32, LANES)

    def compute_group(g, buf, lv):
        def ebody(e, lv):
            ge = g * G + e
            iv = in_idx_v[pl.ds(ge, 16)]
            nv = ng_idx_v[pl.ds(ge * 6, 16)]
            pv0 = pn_idx_v[pl.ds(ge * NL, 16)]
            pv1 = pn_idx_v[pl.ds(ge * NL + 9, 16)]
            off_in = (iv[0] & 1) << 6
            hs = []
            offs_ng = [(nv[q] & 1) << 6 for q in range(6)]
            for k in range(HID // 16):
                acc = ng_rows[buf, e * 6, pl.ds(offs_ng[0] + k * 16, 16)]
                for q in range(1, 6):
                    acc = acc + ng_rows[buf, e * 6 + q,
                                        pl.ds(offs_ng[q] + k * 16, 16)]
                hs.append((in_rows[buf, e, pl.ds(off_in + k * 16, 16)]
                           + acc * jnp.float32(1.0 / 6.0)) * jnp.float32(0.5))
            dv0 = jnp.full((LANES,), 30.0, jnp.float32)
            dv1 = dv0
            for j in range(NL):
                r = e * NL + j
                par = pv0[j] if j < 16 else pv1[j - 9]
                off = (par & 1) << 6
                t = out_rows[buf, r, pl.ds(off, 16)] * hs[0]
                for k in range(1, HID // 16):
                    t = t + out_rows[buf, r, pl.ds(off + k * 16, 16)] * hs[k]
                d = jnp.sum(t)
                db = jnp.full((LANES,), d if j < NP else -d)
                if j < LANES:
                    dv0 = jnp.where(iota == j, db, dv0)
                else:
                    dv1 = jnp.where(iota == (j - LANES), db, dv1)
            l = jnp.sum(_log_sigmoid(dv0)) + jnp.sum(_log_sigmoid(dv1))
            lane = (g % 2) * G + e
            return jnp.where(iota == lane, jnp.full((LANES,), -l), lv)

        return lax.fori_loop(0, G, ebody, lv)

    start_group(0, 0)
    start_group(1, 1)

    def gloop(g, lv):
        buf = lax.rem(g, 3)
        wait_group(g, buf)

        @pl.when(g < NGROUP - 2)
        def _():
            start_group(g + 2, lax.rem(g + 2, 3))

        lv = compute_group(g, buf, lv)
        odd = (g & 1) == 1

        @pl.when(odd)
        def _():
            loss_v[pl.ds(mo((g - 1) * G), LANES)] = lv

        return jnp.where(jnp.full((LANES,), odd),
                         jnp.zeros((LANES,), jnp.float32), lv)

    lax.fori_loop(0, NGROUP, gloop, jnp.zeros((LANES,), jnp.float32))

    pltpu.sync_copy(loss_v, out_hbm.at[pl.ds(base, EPW)])


@jax.jit
def kernel(input_labels, pos_labels, neg_labels, ngram_labels,
           in_emb, ngram_emb, out_emb):
    in_idx = input_labels.astype(jnp.int32)
    ng_idx = ngram_labels.astype(jnp.int32).reshape(-1)
    pn_idx = jnp.concatenate(
        [pos_labels.astype(jnp.int32), neg_labels.astype(jnp.int32)],
        axis=1).reshape(-1)
    mesh = plsc.VectorSubcoreMesh(core_axis_name="c", subcore_axis_name="s")
    line_tab = jax.ShapeDtypeStruct((LINES_PAD, 2 * HID), jnp.float32)
    convert = functools.partial(
        pl.kernel,
        mesh=mesh,
        compiler_params=pltpu.CompilerParams(
            needs_layout_passes=False, use_tc_tiling_on_sc=True),
        out_type=(line_tab, line_tab, line_tab),
        scratch_types=[
            pltpu.VMEM((4, HID, 2 * HID), jnp.float32),
            pltpu.VMEM((4, HID, 2 * HID), jnp.float32),
            pltpu.SemaphoreType.DMA,
            pltpu.SemaphoreType.DMA,
        ],
    )(_transpose_body)
    tails = [t[NBLK * 128:].reshape(16, 2 * HID)
             for t in (in_emb, ngram_emb, out_emb)]
    in_t, ng_t, out_t = convert(in_emb.T, ngram_emb.T, out_emb.T, *tails)
    run = functools.partial(
        pl.kernel,
        mesh=mesh,
        compiler_params=pltpu.CompilerParams(
            needs_layout_passes=False, use_tc_tiling_on_sc=True),
        out_type=jax.ShapeDtypeStruct((B,), jnp.float32),
        scratch_types=[
            pltpu.VMEM((EPW + 16,), jnp.int32),
            pltpu.VMEM((EPW * 6 + 16,), jnp.int32),
            pltpu.VMEM((EPW * NL + 16,), jnp.int32),
            pltpu.VMEM((EPW,), jnp.int32),
            pltpu.VMEM((EPW * 6,), jnp.int32),
            pltpu.VMEM((EPW * NL,), jnp.int32),
            pltpu.VMEM((3, G, 2 * HID), jnp.float32),
            pltpu.VMEM((3, G * 6, 2 * HID), jnp.float32),
            pltpu.VMEM((3, G * NL, 2 * HID), jnp.float32),
            pltpu.VMEM((EPW,), jnp.float32),
            pltpu.SemaphoreType.DMA,
        ],
    )(_sc_body)
    return run(in_idx, ng_idx, pn_idx, in_t, ng_t, out_t)
